# baseline (speedup 1.0000x reference)
import functools

import jax
import jax.numpy as jnp
from jax import lax
from jax.experimental import pallas as pl
from jax.experimental.pallas import tpu as pltpu
from jax.experimental.pallas import tpu_sc as plsc

B, N, C = 2, 25000, 64
BN = B * N
X = Y = Z = 128
GRID = B * X * Y * Z
K = 27
NPAD = 1200
NP = BN + NPAD
PADMASK = 1023

NW = 32
CH = 128
NCHUNK = 13
PW = CH * NCHUNK
PTOT = NW * PW

_OFFS = [(dx, dy, dz) for dx in (-1, 0, 1) for dy in (-1, 0, 1) for dz in (-1, 0, 1)]

MXBLK = 2048
RBLK = 1000


def _h_matmul_body(f_ref, w_ref, o_ref):
    o_ref[...] = jnp.dot(f_ref[...], w_ref[0],
                         preferred_element_type=jnp.float32)


def _h_matmul(feats_pad, w):
    nblk = NP // MXBLK
    return pl.pallas_call(
        _h_matmul_body,
        grid=(nblk, K),
        in_specs=[
            pl.BlockSpec((MXBLK, C), lambda j, k: (j, 0)),
            pl.BlockSpec((1, C, 2 * C), lambda j, k: (k, 0, 0)),
        ],
        out_specs=pl.BlockSpec((MXBLK, 2 * C), lambda j, k: (k * nblk + j, 0)),
        out_shape=jax.ShapeDtypeStruct((K * NP, 2 * C), jnp.float32),
    )(feats_pad, w)


_sc_mesh = plsc.VectorSubcoreMesh(core_axis_name="c", subcore_axis_name="s")


@functools.partial(
    pl.kernel,
    out_type=jax.ShapeDtypeStruct((PTOT, 2 * C), jnp.float32),
    mesh=_sc_mesh,
    scratch_types=[
        pltpu.VMEM((2, CH), jnp.int32),
        pltpu.VMEM((K, CH), jnp.int32),
        pltpu.VMEM((K, CH), jnp.int32),
        pltpu.VMEM((2 * K, CH), jnp.int32),
        pltpu.VMEM((2 * CH, 2 * C), jnp.float32),
        pltpu.SemaphoreType.DMA,
        pltpu.SemaphoreType.DMA,
        pltpu.SemaphoreType.DMA,
        pltpu.SemaphoreType.DMA,
        pltpu.SemaphoreType.DMA,
        pltpu.SemaphoreType.DMA,
        pltpu.SemaphoreType.DMA,
    ],
)
def _sc_conv(flat_hbm, grid_hbm, h_hbm, out_hbm,
             flat_v, nf, gv, ridx, acc, sem_f0, sem_f1, sem_g,
             sem_h0, sem_h1, sem_o0, sem_o1):
    wid = lax.axis_index("s") * 2 + lax.axis_index("c")
    zero16 = jnp.zeros((16,), jnp.float32)
    sem_f = [sem_f0, sem_f1]
    sem_h = [sem_h0, sem_h1]
    sem_o = [sem_o0, sem_o1]

    def prep(ci, par, out_pending):
        base = wid * PW + ci * CH
        pltpu.make_async_copy(flat_hbm.at[pl.ds(base, CH)],
                              flat_v.at[par], sem_f[par]).wait()

        def nf_body(g, carry):
            f = flat_v[par, pl.ds(g * 16, 16)]
            zc = f & 127
            yc = (f >> 7) & 127
            xc = (f >> 14) & 127
            for k, (dx, dy, dz) in enumerate(_OFFS):
                conds = []
                if dx == -1: conds.append(xc >= 1)
                if dx == 1: conds.append(xc <= X - 2)
                if dy == -1: conds.append(yc >= 1)
                if dy == 1: conds.append(yc <= Y - 2)
                if dz == -1: conds.append(zc >= 1)
                if dz == 1: conds.append(zc <= Z - 2)
                nfv = f + (dx * (Y * Z) + dy * Z + dz)
                if conds:
                    m = conds[0]
                    for c2 in conds[1:]:
                        m = m & c2
                    nfv = jnp.where(m, nfv, GRID)
                nf[k, pl.ds(g * 16, 16)] = nfv
            return carry

        lax.fori_loop(0, CH // 16, nf_body, 0)
        pltpu.async_copy(flat_hbm.at[pl.ds(base + 2 * CH, CH)],
                         flat_v.at[par], sem_f[par])
        hg = [pltpu.async_copy(grid_hbm.at[nf.at[k]], gv.at[k], sem_g)
              for k in range(K)]
        for h in hg:
            h.wait()

        def fix_body(g, carry):
            pvec = base + g * 16 + lax.iota(jnp.int32, 16)
            padv = BN + (pvec & PADMASK)
            for k in range(K):
                gval = gv[k, pl.ds(g * 16, 16)]
                ridx[par * K + k, pl.ds(g * 16, 16)] = (
                    jnp.where(gval >= 0, gval, padv) + k * NP)
            return carry

        lax.fori_loop(0, CH // 16, fix_body, 0)
        if out_pending:
            pltpu.make_async_copy(out_hbm.at[pl.ds(0, CH)],
                                  acc.at[pl.ds(par * CH, CH)], sem_o[par]).wait()

        def zero_body(r, carry):
            for cb in range(2 * C // 16):
                acc[par * CH + r, pl.ds(cb * 16, 16)] = zero16
            return carry

        lax.fori_loop(0, CH, zero_body, 0)
        for k in range(K):
            pltpu.async_copy(h_hbm.at[ridx.at[par * K + k]],
                             acc.at[pl.ds(par * CH, CH)], sem_h[par], add=True)
        return base

    def fin(ci, par):
        base = wid * PW + ci * CH
        for _ in range(K):
            pltpu.make_async_copy(h_hbm.at[pl.ds(0, CH)],
                                  acc.at[pl.ds(par * CH, CH)], sem_h[par]).wait()
        pltpu.async_copy(acc.at[pl.ds(par * CH, CH)],
                         out_hbm.at[pl.ds(base, CH)], sem_o[par])

    pltpu.async_copy(flat_hbm.at[pl.ds(wid * PW, CH)], flat_v.at[0], sem_f0)
    pltpu.async_copy(flat_hbm.at[pl.ds(wid * PW + CH, CH)], flat_v.at[1], sem_f1)
    prep(0, 0, False)

    def loop_body(i, carry):
        c0 = 2 * i

        @pl.when(i >= 1)
        def _():
            pltpu.make_async_copy(out_hbm.at[pl.ds(0, CH)],
                                  acc.at[pl.ds(CH, CH)], sem_o1).wait()
        prep(c0 + 1, 1, False)
        fin(c0, 0)
        prep(c0 + 2, 0, True)
        fin(c0 + 1, 1)
        return carry

    lax.fori_loop(0, (NCHUNK - 1) // 2, loop_body, 0)
    fin(NCHUNK - 1, 0)
    pltpu.make_async_copy(out_hbm.at[pl.ds(0, CH)],
                          acc.at[pl.ds(0, CH)], sem_o0).wait()
    pltpu.make_async_copy(out_hbm.at[pl.ds(0, CH)],
                          acc.at[pl.ds(CH, CH)], sem_o1).wait()
    pltpu.make_async_copy(flat_hbm.at[pl.ds(0, CH)],
                          flat_v.at[0], sem_f0).wait()
    pltpu.make_async_copy(flat_hbm.at[pl.ds(0, CH)],
                          flat_v.at[1], sem_f1).wait()


def _bn_reduce_body(x_ref, s_ref, q_ref):
    i = pl.program_id(0)
    xs = x_ref[:, 0:C].astype(jnp.float32)
    s = jnp.broadcast_to(jnp.sum(xs, axis=0, keepdims=True), (8, C))
    q = jnp.broadcast_to(jnp.sum(xs * xs, axis=0, keepdims=True), (8, C))

    @pl.when(i == 0)
    def _():
        s_ref[...] = s
        q_ref[...] = q

    @pl.when(i > 0)
    def _():
        s_ref[...] += s
        q_ref[...] += q


def _bn_reduce(out_pre):
    return pl.pallas_call(
        _bn_reduce_body,
        grid=(BN // RBLK,),
        in_specs=[pl.BlockSpec((RBLK, 2 * C), lambda i: (i, 0))],
        out_specs=[pl.BlockSpec((8, C), lambda i: (0, 0))] * 2,
        out_shape=[jax.ShapeDtypeStruct((8, C), jnp.float32)] * 2,
    )(out_pre)


def _bn_norm_body(x_ref, s_ref, q_ref, g_ref, b_ref, o_ref):
    mean = s_ref[0:1, :] * (1.0 / BN)
    var = q_ref[0:1, :] * (1.0 / BN) - mean * mean
    inv = lax.rsqrt(var + 1e-5)
    scale = g_ref[0:1, :] * inv
    shift = b_ref[0:1, :] - mean * scale
    o_ref[...] = jnp.maximum(
        x_ref[:, 0:C].astype(jnp.float32) * scale + shift, 0.0)


def _bn_norm(out_pre, s, q, gamma8, beta8):
    return pl.pallas_call(
        _bn_norm_body,
        grid=(BN // RBLK,),
        in_specs=[
            pl.BlockSpec((RBLK, 2 * C), lambda i: (i, 0)),
            pl.BlockSpec((8, C), lambda i: (0, 0)),
            pl.BlockSpec((8, C), lambda i: (0, 0)),
            pl.BlockSpec((8, C), lambda i: (0, 0)),
            pl.BlockSpec((8, C), lambda i: (0, 0)),
        ],
        out_specs=pl.BlockSpec((RBLK, C), lambda i: (i, 0)),
        out_shape=jax.ShapeDtypeStruct((BN, C), jnp.float32),
    )(out_pre, s, q, gamma8, beta8)


def kernel(features, coordinates, spatial_shape, batch_size, W, gamma, beta):
    feats = features.reshape(BN, C)
    coords = coordinates.reshape(BN, 3).astype(jnp.int32)
    bidx = jnp.repeat(jnp.arange(B, dtype=jnp.int32), N)
    flat = ((bidx * X + coords[:, 0]) * Y + coords[:, 1]) * Z + coords[:, 2]
    grid_ext = jnp.full((GRID + 1,), -1, dtype=jnp.int32).at[flat].set(
        jnp.arange(BN, dtype=jnp.int32))
    feats_pad = jnp.concatenate(
        [feats, jnp.zeros((NPAD, C), feats.dtype)], axis=0)
    w128 = jnp.concatenate([W, jnp.zeros((K, C, C), W.dtype)], axis=2)
    h = _h_matmul(feats_pad, w128)
    flat_pad = jnp.concatenate(
        [flat, jnp.zeros((PTOT + 2 * CH - BN,), jnp.int32)], axis=0)
    out_pre = _sc_conv(flat_pad, grid_ext, h)
    s, q = _bn_reduce(out_pre)
    gamma8 = jnp.broadcast_to(gamma.reshape(1, C), (8, C))
    beta8 = jnp.broadcast_to(beta.reshape(1, C), (8, C))
    y = _bn_norm(out_pre, s, q, gamma8, beta8)
    return y.reshape(B, N, C)


# speedup vs baseline: 1.0017x; 1.0017x over previous
import functools

import jax
import jax.numpy as jnp
from jax import lax
from jax.experimental import pallas as pl
from jax.experimental.pallas import tpu as pltpu
from jax.experimental.pallas import tpu_sc as plsc

B, N, C = 2, 25000, 64
BN = B * N
X = Y = Z = 128
GRID = B * X * Y * Z
K = 27
NPAD = 1200
NP = BN + NPAD
PADMASK = 1023

NW = 32
CH = 128
NCHUNK = 13
PW = CH * NCHUNK
PTOT = NW * PW

_OFFS = [(dx, dy, dz) for dx in (-1, 0, 1) for dy in (-1, 0, 1) for dz in (-1, 0, 1)]

MXBLK = 2048
RBLK = 1000


def _h_matmul_body(f_ref, w_ref, o_ref):
    o_ref[...] = jnp.dot(f_ref[...], w_ref[0],
                         preferred_element_type=jnp.float32)


def _h_matmul(feats_pad, w):
    nblk = NP // MXBLK
    return pl.pallas_call(
        _h_matmul_body,
        grid=(nblk, K),
        in_specs=[
            pl.BlockSpec((MXBLK, C), lambda j, k: (j, 0)),
            pl.BlockSpec((1, C, 2 * C), lambda j, k: (k, 0, 0)),
        ],
        out_specs=pl.BlockSpec((MXBLK, 2 * C), lambda j, k: (k * nblk + j, 0)),
        out_shape=jax.ShapeDtypeStruct((K * NP, 2 * C), jnp.float32),
    )(feats_pad, w)


_sc_mesh = plsc.VectorSubcoreMesh(core_axis_name="c", subcore_axis_name="s")


@functools.partial(
    pl.kernel,
    out_type=jax.ShapeDtypeStruct((PTOT, 2 * C), jnp.float32),
    mesh=_sc_mesh,
    scratch_types=[
        pltpu.VMEM((2, CH), jnp.int32),
        pltpu.VMEM((K * CH,), jnp.int32),
        pltpu.VMEM((K * CH,), jnp.int32),
        pltpu.VMEM((2 * K, CH), jnp.int32),
        pltpu.VMEM((2 * CH, 2 * C), jnp.float32),
        pltpu.SemaphoreType.DMA,
        pltpu.SemaphoreType.DMA,
        pltpu.SemaphoreType.DMA,
        pltpu.SemaphoreType.DMA,
        pltpu.SemaphoreType.DMA,
        pltpu.SemaphoreType.DMA,
        pltpu.SemaphoreType.DMA,
    ],
)
def _sc_conv(flat_hbm, grid_hbm, h_hbm, out_hbm,
             flat_v, nf, gv, ridx, acc, sem_f0, sem_f1, sem_g,
             sem_h0, sem_h1, sem_o0, sem_o1):
    wid = lax.axis_index("s") * 2 + lax.axis_index("c")
    zero16 = jnp.zeros((16,), jnp.float32)
    sem_f = [sem_f0, sem_f1]
    sem_h = [sem_h0, sem_h1]
    sem_o = [sem_o0, sem_o1]

    def prep(ci, par, out_pending):
        base = wid * PW + ci * CH
        pltpu.make_async_copy(flat_hbm.at[pl.ds(base, CH)],
                              flat_v.at[par], sem_f[par]).wait()

        def nf_body(g, carry):
            f = flat_v[par, pl.ds(g * 16, 16)]
            zc = f & 127
            yc = (f >> 7) & 127
            xc = (f >> 14) & 127
            for k, (dx, dy, dz) in enumerate(_OFFS):
                conds = []
                if dx == -1: conds.append(xc >= 1)
                if dx == 1: conds.append(xc <= X - 2)
                if dy == -1: conds.append(yc >= 1)
                if dy == 1: conds.append(yc <= Y - 2)
                if dz == -1: conds.append(zc >= 1)
                if dz == 1: conds.append(zc <= Z - 2)
                nfv = f + (dx * (Y * Z) + dy * Z + dz)
                if conds:
                    m = conds[0]
                    for c2 in conds[1:]:
                        m = m & c2
                    nfv = jnp.where(m, nfv, GRID)
                nf[pl.ds(k * CH + g * 16, 16)] = nfv
            return carry

        lax.fori_loop(0, CH // 16, nf_body, 0)
        pltpu.async_copy(flat_hbm.at[pl.ds(base + 2 * CH, CH)],
                         flat_v.at[par], sem_f[par])
        pltpu.async_copy(grid_hbm.at[nf], gv, sem_g).wait()

        def fix_body(g, carry):
            pvec = base + g * 16 + lax.iota(jnp.int32, 16)
            padv = BN + (pvec & PADMASK)
            for k in range(K):
                gval = gv[pl.ds(k * CH + g * 16, 16)]
                ridx[par * K + k, pl.ds(g * 16, 16)] = (
                    jnp.where(gval >= 0, gval, padv) + k * NP)
            return carry

        lax.fori_loop(0, CH // 16, fix_body, 0)
        if out_pending:
            pltpu.make_async_copy(out_hbm.at[pl.ds(0, CH)],
                                  acc.at[pl.ds(par * CH, CH)], sem_o[par]).wait()

        def zero_body(r, carry):
            for cb in range(2 * C // 16):
                acc[par * CH + r, pl.ds(cb * 16, 16)] = zero16
            return carry

        lax.fori_loop(0, CH, zero_body, 0)
        for k in range(K):
            pltpu.async_copy(h_hbm.at[ridx.at[par * K + k]],
                             acc.at[pl.ds(par * CH, CH)], sem_h[par], add=True)
        return base

    def fin(ci, par):
        base = wid * PW + ci * CH
        for _ in range(K):
            pltpu.make_async_copy(h_hbm.at[pl.ds(0, CH)],
                                  acc.at[pl.ds(par * CH, CH)], sem_h[par]).wait()
        pltpu.async_copy(acc.at[pl.ds(par * CH, CH)],
                         out_hbm.at[pl.ds(base, CH)], sem_o[par])

    pltpu.async_copy(flat_hbm.at[pl.ds(wid * PW, CH)], flat_v.at[0], sem_f0)
    pltpu.async_copy(flat_hbm.at[pl.ds(wid * PW + CH, CH)], flat_v.at[1], sem_f1)
    prep(0, 0, False)

    def loop_body(i, carry):
        c0 = 2 * i

        @pl.when(i >= 1)
        def _():
            pltpu.make_async_copy(out_hbm.at[pl.ds(0, CH)],
                                  acc.at[pl.ds(CH, CH)], sem_o1).wait()
        prep(c0 + 1, 1, False)
        fin(c0, 0)
        prep(c0 + 2, 0, True)
        fin(c0 + 1, 1)
        return carry

    lax.fori_loop(0, (NCHUNK - 1) // 2, loop_body, 0)
    fin(NCHUNK - 1, 0)
    pltpu.make_async_copy(out_hbm.at[pl.ds(0, CH)],
                          acc.at[pl.ds(0, CH)], sem_o0).wait()
    pltpu.make_async_copy(out_hbm.at[pl.ds(0, CH)],
                          acc.at[pl.ds(CH, CH)], sem_o1).wait()
    pltpu.make_async_copy(flat_hbm.at[pl.ds(0, CH)],
                          flat_v.at[0], sem_f0).wait()
    pltpu.make_async_copy(flat_hbm.at[pl.ds(0, CH)],
                          flat_v.at[1], sem_f1).wait()


def _bn_reduce_body(x_ref, s_ref, q_ref):
    i = pl.program_id(0)
    xs = x_ref[:, 0:C].astype(jnp.float32)
    s = jnp.broadcast_to(jnp.sum(xs, axis=0, keepdims=True), (8, C))
    q = jnp.broadcast_to(jnp.sum(xs * xs, axis=0, keepdims=True), (8, C))

    @pl.when(i == 0)
    def _():
        s_ref[...] = s
        q_ref[...] = q

    @pl.when(i > 0)
    def _():
        s_ref[...] += s
        q_ref[...] += q


def _bn_reduce(out_pre):
    return pl.pallas_call(
        _bn_reduce_body,
        grid=(BN // RBLK,),
        in_specs=[pl.BlockSpec((RBLK, 2 * C), lambda i: (i, 0))],
        out_specs=[pl.BlockSpec((8, C), lambda i: (0, 0))] * 2,
        out_shape=[jax.ShapeDtypeStruct((8, C), jnp.float32)] * 2,
    )(out_pre)


def _bn_norm_body(x_ref, s_ref, q_ref, g_ref, b_ref, o_ref):
    mean = s_ref[0:1, :] * (1.0 / BN)
    var = q_ref[0:1, :] * (1.0 / BN) - mean * mean
    inv = lax.rsqrt(var + 1e-5)
    scale = g_ref[0:1, :] * inv
    shift = b_ref[0:1, :] - mean * scale
    o_ref[...] = jnp.maximum(
        x_ref[:, 0:C].astype(jnp.float32) * scale + shift, 0.0)


def _bn_norm(out_pre, s, q, gamma8, beta8):
    return pl.pallas_call(
        _bn_norm_body,
        grid=(BN // RBLK,),
        in_specs=[
            pl.BlockSpec((RBLK, 2 * C), lambda i: (i, 0)),
            pl.BlockSpec((8, C), lambda i: (0, 0)),
            pl.BlockSpec((8, C), lambda i: (0, 0)),
            pl.BlockSpec((8, C), lambda i: (0, 0)),
            pl.BlockSpec((8, C), lambda i: (0, 0)),
        ],
        out_specs=pl.BlockSpec((RBLK, C), lambda i: (i, 0)),
        out_shape=jax.ShapeDtypeStruct((BN, C), jnp.float32),
    )(out_pre, s, q, gamma8, beta8)


def kernel(features, coordinates, spatial_shape, batch_size, W, gamma, beta):
    feats = features.reshape(BN, C)
    coords = coordinates.reshape(BN, 3).astype(jnp.int32)
    bidx = jnp.repeat(jnp.arange(B, dtype=jnp.int32), N)
    flat = ((bidx * X + coords[:, 0]) * Y + coords[:, 1]) * Z + coords[:, 2]
    grid_ext = jnp.full((GRID + 1,), -1, dtype=jnp.int32).at[flat].set(
        jnp.arange(BN, dtype=jnp.int32))
    feats_pad = jnp.concatenate(
        [feats, jnp.zeros((NPAD, C), feats.dtype)], axis=0)
    w128 = jnp.concatenate([W, jnp.zeros((K, C, C), W.dtype)], axis=2)
    h = _h_matmul(feats_pad, w128)
    flat_pad = jnp.concatenate(
        [flat, jnp.zeros((PTOT + 2 * CH - BN,), jnp.int32)], axis=0)
    out_pre = _sc_conv(flat_pad, grid_ext, h)
    s, q = _bn_reduce(out_pre)
    gamma8 = jnp.broadcast_to(gamma.reshape(1, C), (8, C))
    beta8 = jnp.broadcast_to(beta.reshape(1, C), (8, C))
    y = _bn_norm(out_pre, s, q, gamma8, beta8)
    return y.reshape(B, N, C)


# speedup vs baseline: 1.0473x; 1.0455x over previous
import functools

import jax
import jax.numpy as jnp
from jax import lax
from jax.experimental import pallas as pl
from jax.experimental.pallas import tpu as pltpu
from jax.experimental.pallas import tpu_sc as plsc

B, N, C = 2, 25000, 64
BN = B * N
X = Y = Z = 128
GRID = B * X * Y * Z
K = 27
NPAD = 1200
NP = BN + NPAD
PADMASK = 1023

NW = 32
CH = 128
NCHUNK = 13
PW = CH * NCHUNK
PTOT = NW * PW

_OFFS = [(dx, dy, dz) for dx in (-1, 0, 1) for dy in (-1, 0, 1) for dz in (-1, 0, 1)]

MXBLK = 2560
RBLK = 1000


def _h_matmul_body(f_ref, w_ref, o_ref):
    o_ref[...] = jnp.dot(f_ref[...], w_ref[0],
                         preferred_element_type=jnp.float32)


def _h_matmul(feats_pad, w):
    nblk = NP // MXBLK
    return pl.pallas_call(
        _h_matmul_body,
        grid=(nblk, K),
        in_specs=[
            pl.BlockSpec((MXBLK, C), lambda j, k: (j, 0)),
            pl.BlockSpec((1, C, 2 * C), lambda j, k: (k, 0, 0)),
        ],
        out_specs=pl.BlockSpec((MXBLK, 2 * C), lambda j, k: (k * nblk + j, 0)),
        out_shape=jax.ShapeDtypeStruct((K * NP, 2 * C), jnp.float32),
    )(feats_pad, w)


_sc_mesh = plsc.VectorSubcoreMesh(core_axis_name="c", subcore_axis_name="s")


@functools.partial(
    pl.kernel,
    out_type=jax.ShapeDtypeStruct((PTOT, 2 * C), jnp.float32),
    mesh=_sc_mesh,
    scratch_types=[
        pltpu.VMEM((2, CH), jnp.int32),
        pltpu.VMEM((K * CH,), jnp.int32),
        pltpu.VMEM((K * CH,), jnp.int32),
        pltpu.VMEM((2 * K, CH), jnp.int32),
        pltpu.VMEM((2 * CH, 2 * C), jnp.float32),
        pltpu.SemaphoreType.DMA,
        pltpu.SemaphoreType.DMA,
        pltpu.SemaphoreType.DMA,
        pltpu.SemaphoreType.DMA,
        pltpu.SemaphoreType.DMA,
        pltpu.SemaphoreType.DMA,
        pltpu.SemaphoreType.DMA,
    ],
)
def _sc_conv(flat_hbm, grid_hbm, h_hbm, out_hbm,
             flat_v, nf, gv, ridx, acc, sem_f0, sem_f1, sem_g,
             sem_h0, sem_h1, sem_o0, sem_o1):
    wid = lax.axis_index("s") * 2 + lax.axis_index("c")
    zero16 = jnp.zeros((16,), jnp.float32)
    sem_f = [sem_f0, sem_f1]
    sem_h = [sem_h0, sem_h1]
    sem_o = [sem_o0, sem_o1]

    def prep(ci, par, out_pending):
        base = wid * PW + ci * CH
        pltpu.make_async_copy(flat_hbm.at[pl.ds(base, CH)],
                              flat_v.at[par], sem_f[par]).wait()

        def nf_body(g, carry):
            f = flat_v[par, pl.ds(g * 16, 16)]
            zc = f & 127
            yc = (f >> 7) & 127
            xc = (f >> 14) & 127
            for k, (dx, dy, dz) in enumerate(_OFFS):
                conds = []
                if dx == -1: conds.append(xc >= 1)
                if dx == 1: conds.append(xc <= X - 2)
                if dy == -1: conds.append(yc >= 1)
                if dy == 1: conds.append(yc <= Y - 2)
                if dz == -1: conds.append(zc >= 1)
                if dz == 1: conds.append(zc <= Z - 2)
                nfv = f + (dx * (Y * Z) + dy * Z + dz)
                if conds:
                    m = conds[0]
                    for c2 in conds[1:]:
                        m = m & c2
                    nfv = jnp.where(m, nfv, GRID)
                nf[pl.ds(k * CH + g * 16, 16)] = nfv
            return carry

        lax.fori_loop(0, CH // 16, nf_body, 0)
        pltpu.async_copy(flat_hbm.at[pl.ds(base + 2 * CH, CH)],
                         flat_v.at[par], sem_f[par])
        pltpu.async_copy(grid_hbm.at[nf], gv, sem_g).wait()

        def fix_body(g, carry):
            pvec = base + g * 16 + lax.iota(jnp.int32, 16)
            padv = BN + (pvec & PADMASK)
            for k in range(K):
                gval = gv[pl.ds(k * CH + g * 16, 16)]
                ridx[par * K + k, pl.ds(g * 16, 16)] = (
                    jnp.where(gval >= 0, gval, padv) + k * NP)
            return carry

        lax.fori_loop(0, CH // 16, fix_body, 0)
        if out_pending:
            pltpu.make_async_copy(out_hbm.at[pl.ds(0, CH)],
                                  acc.at[pl.ds(par * CH, CH)], sem_o[par]).wait()

        def zero_body(r, carry):
            for cb in range(2 * C // 16):
                acc[par * CH + r, pl.ds(cb * 16, 16)] = zero16
            return carry

        lax.fori_loop(0, CH, zero_body, 0)
        for k in range(K):
            pltpu.async_copy(h_hbm.at[ridx.at[par * K + k]],
                             acc.at[pl.ds(par * CH, CH)], sem_h[par], add=True)
        return base

    def fin(ci, par):
        base = wid * PW + ci * CH
        for _ in range(K):
            pltpu.make_async_copy(h_hbm.at[pl.ds(0, CH)],
                                  acc.at[pl.ds(par * CH, CH)], sem_h[par]).wait()
        pltpu.async_copy(acc.at[pl.ds(par * CH, CH)],
                         out_hbm.at[pl.ds(base, CH)], sem_o[par])

    pltpu.async_copy(flat_hbm.at[pl.ds(wid * PW, CH)], flat_v.at[0], sem_f0)
    pltpu.async_copy(flat_hbm.at[pl.ds(wid * PW + CH, CH)], flat_v.at[1], sem_f1)
    prep(0, 0, False)

    def loop_body(i, carry):
        c0 = 2 * i

        @pl.when(i >= 1)
        def _():
            pltpu.make_async_copy(out_hbm.at[pl.ds(0, CH)],
                                  acc.at[pl.ds(CH, CH)], sem_o1).wait()
        prep(c0 + 1, 1, False)
        fin(c0, 0)
        prep(c0 + 2, 0, True)
        fin(c0 + 1, 1)
        return carry

    lax.fori_loop(0, (NCHUNK - 1) // 2, loop_body, 0)
    fin(NCHUNK - 1, 0)
    pltpu.make_async_copy(out_hbm.at[pl.ds(0, CH)],
                          acc.at[pl.ds(0, CH)], sem_o0).wait()
    pltpu.make_async_copy(out_hbm.at[pl.ds(0, CH)],
                          acc.at[pl.ds(CH, CH)], sem_o1).wait()
    pltpu.make_async_copy(flat_hbm.at[pl.ds(0, CH)],
                          flat_v.at[0], sem_f0).wait()
    pltpu.make_async_copy(flat_hbm.at[pl.ds(0, CH)],
                          flat_v.at[1], sem_f1).wait()


def _bn_reduce_body(x_ref, s_ref, q_ref):
    i = pl.program_id(0)
    xs = x_ref[:, 0:C].astype(jnp.float32)
    s = jnp.broadcast_to(jnp.sum(xs, axis=0, keepdims=True), (8, C))
    q = jnp.broadcast_to(jnp.sum(xs * xs, axis=0, keepdims=True), (8, C))

    @pl.when(i == 0)
    def _():
        s_ref[...] = s
        q_ref[...] = q

    @pl.when(i > 0)
    def _():
        s_ref[...] += s
        q_ref[...] += q


def _bn_reduce(out_pre):
    return pl.pallas_call(
        _bn_reduce_body,
        grid=(BN // RBLK,),
        in_specs=[pl.BlockSpec((RBLK, 2 * C), lambda i: (i, 0))],
        out_specs=[pl.BlockSpec((8, C), lambda i: (0, 0))] * 2,
        out_shape=[jax.ShapeDtypeStruct((8, C), jnp.float32)] * 2,
    )(out_pre)


def _bn_norm_body(x_ref, s_ref, q_ref, g_ref, b_ref, o_ref):
    mean = s_ref[0:1, :] * (1.0 / BN)
    var = q_ref[0:1, :] * (1.0 / BN) - mean * mean
    inv = lax.rsqrt(var + 1e-5)
    scale = g_ref[0:1, :] * inv
    shift = b_ref[0:1, :] - mean * scale
    o_ref[...] = jnp.maximum(
        x_ref[:, 0:C].astype(jnp.float32) * scale + shift, 0.0)


def _bn_norm(out_pre, s, q, gamma8, beta8):
    return pl.pallas_call(
        _bn_norm_body,
        grid=(BN // RBLK,),
        in_specs=[
            pl.BlockSpec((RBLK, 2 * C), lambda i: (i, 0)),
            pl.BlockSpec((8, C), lambda i: (0, 0)),
            pl.BlockSpec((8, C), lambda i: (0, 0)),
            pl.BlockSpec((8, C), lambda i: (0, 0)),
            pl.BlockSpec((8, C), lambda i: (0, 0)),
        ],
        out_specs=pl.BlockSpec((RBLK, C), lambda i: (i, 0)),
        out_shape=jax.ShapeDtypeStruct((BN, C), jnp.float32),
    )(out_pre, s, q, gamma8, beta8)


def kernel(features, coordinates, spatial_shape, batch_size, W, gamma, beta):
    feats = features.reshape(BN, C)
    coords = coordinates.reshape(BN, 3).astype(jnp.int32)
    bidx = jnp.repeat(jnp.arange(B, dtype=jnp.int32), N)
    flat = ((bidx * X + coords[:, 0]) * Y + coords[:, 1]) * Z + coords[:, 2]
    grid_ext = jnp.full((GRID + 1,), -1, dtype=jnp.int32).at[flat].set(
        jnp.arange(BN, dtype=jnp.int32))
    feats_pad = jnp.concatenate(
        [feats, jnp.zeros((NPAD, C), feats.dtype)], axis=0)
    w128 = jnp.concatenate([W, jnp.zeros((K, C, C), W.dtype)], axis=2)
    h = _h_matmul(feats_pad, w128)
    flat_pad = jnp.concatenate(
        [flat, jnp.zeros((PTOT + 2 * CH - BN,), jnp.int32)], axis=0)
    out_pre = _sc_conv(flat_pad, grid_ext, h)
    s, q = _bn_reduce(out_pre)
    gamma8 = jnp.broadcast_to(gamma.reshape(1, C), (8, C))
    beta8 = jnp.broadcast_to(beta.reshape(1, C), (8, C))
    y = _bn_norm(out_pre, s, q, gamma8, beta8)
    return y.reshape(B, N, C)


# speedup vs baseline: 1.1680x; 1.1153x over previous
import functools

import jax
import jax.numpy as jnp
from jax import lax
from jax.experimental import pallas as pl
from jax.experimental.pallas import tpu as pltpu
from jax.experimental.pallas import tpu_sc as plsc

B, N, C = 2, 25000, 64
BN = B * N
X = Y = Z = 128
GRID = B * X * Y * Z
K = 27
NPAD = 1200
NP = BN + NPAD
PADMASK = 1023

NW = 32
CH = 128
NCHUNK = 13
PW = CH * NCHUNK
PTOT = NW * PW

_OFFS = [(dx, dy, dz) for dx in (-1, 0, 1) for dy in (-1, 0, 1) for dz in (-1, 0, 1)]

MXBLK = 5120
RBLK = 1000


def _h_matmul_body(f_ref, w_ref, o_ref):
    o_ref[...] = jnp.dot(f_ref[...], w_ref[0],
                         preferred_element_type=jnp.float32)


def _h_matmul(feats_pad, w):
    nblk = NP // MXBLK
    return pl.pallas_call(
        _h_matmul_body,
        grid=(nblk, K),
        in_specs=[
            pl.BlockSpec((MXBLK, C), lambda j, k: (j, 0)),
            pl.BlockSpec((1, C, 2 * C), lambda j, k: (k, 0, 0)),
        ],
        out_specs=pl.BlockSpec((MXBLK, 2 * C), lambda j, k: (k * nblk + j, 0)),
        out_shape=jax.ShapeDtypeStruct((K * NP, 2 * C), jnp.float32),
    )(feats_pad, w)


_sc_mesh = plsc.VectorSubcoreMesh(core_axis_name="c", subcore_axis_name="s")


@functools.partial(
    pl.kernel,
    out_type=jax.ShapeDtypeStruct((PTOT, 2 * C), jnp.float32),
    mesh=_sc_mesh,
    scratch_types=[
        pltpu.VMEM((2, CH), jnp.int32),
        pltpu.VMEM((K * CH,), jnp.int32),
        pltpu.VMEM((K * CH,), jnp.int32),
        pltpu.VMEM((2 * K, CH), jnp.int32),
        pltpu.VMEM((2 * CH, 2 * C), jnp.float32),
        pltpu.SemaphoreType.DMA,
        pltpu.SemaphoreType.DMA,
        pltpu.SemaphoreType.DMA,
        pltpu.SemaphoreType.DMA,
        pltpu.SemaphoreType.DMA,
        pltpu.SemaphoreType.DMA,
        pltpu.SemaphoreType.DMA,
    ],
)
def _sc_conv(flat_hbm, grid_hbm, h_hbm, out_hbm,
             flat_v, nf, gv, ridx, acc, sem_f0, sem_f1, sem_g,
             sem_h0, sem_h1, sem_o0, sem_o1):
    wid = lax.axis_index("s") * 2 + lax.axis_index("c")
    zero16 = jnp.zeros((16,), jnp.float32)
    sem_f = [sem_f0, sem_f1]
    sem_h = [sem_h0, sem_h1]
    sem_o = [sem_o0, sem_o1]

    def prep(ci, par, out_pending):
        base = wid * PW + ci * CH
        pltpu.make_async_copy(flat_hbm.at[pl.ds(base, CH)],
                              flat_v.at[par], sem_f[par]).wait()

        def nf_body(g, carry):
            f = flat_v[par, pl.ds(g * 16, 16)]
            zc = f & 127
            yc = (f >> 7) & 127
            xc = (f >> 14) & 127
            for k, (dx, dy, dz) in enumerate(_OFFS):
                conds = []
                if dx == -1: conds.append(xc >= 1)
                if dx == 1: conds.append(xc <= X - 2)
                if dy == -1: conds.append(yc >= 1)
                if dy == 1: conds.append(yc <= Y - 2)
                if dz == -1: conds.append(zc >= 1)
                if dz == 1: conds.append(zc <= Z - 2)
                nfv = f + (dx * (Y * Z) + dy * Z + dz)
                if conds:
                    m = conds[0]
                    for c2 in conds[1:]:
                        m = m & c2
                    nfv = jnp.where(m, nfv, GRID)
                nf[pl.ds(k * CH + g * 16, 16)] = nfv
            return carry

        lax.fori_loop(0, CH // 16, nf_body, 0)
        pltpu.async_copy(flat_hbm.at[pl.ds(base + 2 * CH, CH)],
                         flat_v.at[par], sem_f[par])
        pltpu.async_copy(grid_hbm.at[nf], gv, sem_g).wait()

        def fix_body(g, carry):
            pvec = base + g * 16 + lax.iota(jnp.int32, 16)
            padv = BN + (pvec & PADMASK)
            for k in range(K):
                gval = gv[pl.ds(k * CH + g * 16, 16)]
                ridx[par * K + k, pl.ds(g * 16, 16)] = (
                    jnp.where(gval >= 0, gval, padv) + k * NP)
            return carry

        lax.fori_loop(0, CH // 16, fix_body, 0)
        if out_pending:
            pltpu.make_async_copy(out_hbm.at[pl.ds(0, CH)],
                                  acc.at[pl.ds(par * CH, CH)], sem_o[par]).wait()

        def zero_body(r, carry):
            for cb in range(2 * C // 16):
                acc[par * CH + r, pl.ds(cb * 16, 16)] = zero16
            return carry

        lax.fori_loop(0, CH, zero_body, 0)
        for k in range(K):
            pltpu.async_copy(h_hbm.at[ridx.at[par * K + k]],
                             acc.at[pl.ds(par * CH, CH)], sem_h[par], add=True)
        return base

    def fin(ci, par):
        base = wid * PW + ci * CH
        for _ in range(K):
            pltpu.make_async_copy(h_hbm.at[pl.ds(0, CH)],
                                  acc.at[pl.ds(par * CH, CH)], sem_h[par]).wait()
        pltpu.async_copy(acc.at[pl.ds(par * CH, CH)],
                         out_hbm.at[pl.ds(base, CH)], sem_o[par])

    pltpu.async_copy(flat_hbm.at[pl.ds(wid * PW, CH)], flat_v.at[0], sem_f0)
    pltpu.async_copy(flat_hbm.at[pl.ds(wid * PW + CH, CH)], flat_v.at[1], sem_f1)
    prep(0, 0, False)

    def loop_body(i, carry):
        c0 = 2 * i

        @pl.when(i >= 1)
        def _():
            pltpu.make_async_copy(out_hbm.at[pl.ds(0, CH)],
                                  acc.at[pl.ds(CH, CH)], sem_o1).wait()
        prep(c0 + 1, 1, False)
        fin(c0, 0)
        prep(c0 + 2, 0, True)
        fin(c0 + 1, 1)
        return carry

    lax.fori_loop(0, (NCHUNK - 1) // 2, loop_body, 0)
    fin(NCHUNK - 1, 0)
    pltpu.make_async_copy(out_hbm.at[pl.ds(0, CH)],
                          acc.at[pl.ds(0, CH)], sem_o0).wait()
    pltpu.make_async_copy(out_hbm.at[pl.ds(0, CH)],
                          acc.at[pl.ds(CH, CH)], sem_o1).wait()
    pltpu.make_async_copy(flat_hbm.at[pl.ds(0, CH)],
                          flat_v.at[0], sem_f0).wait()
    pltpu.make_async_copy(flat_hbm.at[pl.ds(0, CH)],
                          flat_v.at[1], sem_f1).wait()


def _bn_reduce_body(x_ref, s_ref, q_ref):
    i = pl.program_id(0)
    xs = x_ref[:, 0:C].astype(jnp.float32)
    s = jnp.broadcast_to(jnp.sum(xs, axis=0, keepdims=True), (8, C))
    q = jnp.broadcast_to(jnp.sum(xs * xs, axis=0, keepdims=True), (8, C))

    @pl.when(i == 0)
    def _():
        s_ref[...] = s
        q_ref[...] = q

    @pl.when(i > 0)
    def _():
        s_ref[...] += s
        q_ref[...] += q


def _bn_reduce(out_pre):
    return pl.pallas_call(
        _bn_reduce_body,
        grid=(BN // RBLK,),
        in_specs=[pl.BlockSpec((RBLK, 2 * C), lambda i: (i, 0))],
        out_specs=[pl.BlockSpec((8, C), lambda i: (0, 0))] * 2,
        out_shape=[jax.ShapeDtypeStruct((8, C), jnp.float32)] * 2,
    )(out_pre)


def _bn_norm_body(x_ref, s_ref, q_ref, g_ref, b_ref, o_ref):
    mean = s_ref[0:1, :] * (1.0 / BN)
    var = q_ref[0:1, :] * (1.0 / BN) - mean * mean
    inv = lax.rsqrt(var + 1e-5)
    scale = g_ref[0:1, :] * inv
    shift = b_ref[0:1, :] - mean * scale
    o_ref[...] = jnp.maximum(
        x_ref[:, 0:C].astype(jnp.float32) * scale + shift, 0.0)


def _bn_norm(out_pre, s, q, gamma8, beta8):
    return pl.pallas_call(
        _bn_norm_body,
        grid=(BN // RBLK,),
        in_specs=[
            pl.BlockSpec((RBLK, 2 * C), lambda i: (i, 0)),
            pl.BlockSpec((8, C), lambda i: (0, 0)),
            pl.BlockSpec((8, C), lambda i: (0, 0)),
            pl.BlockSpec((8, C), lambda i: (0, 0)),
            pl.BlockSpec((8, C), lambda i: (0, 0)),
        ],
        out_specs=pl.BlockSpec((RBLK, C), lambda i: (i, 0)),
        out_shape=jax.ShapeDtypeStruct((BN, C), jnp.float32),
    )(out_pre, s, q, gamma8, beta8)


def kernel(features, coordinates, spatial_shape, batch_size, W, gamma, beta):
    feats = features.reshape(BN, C)
    coords = coordinates.reshape(BN, 3).astype(jnp.int32)
    bidx = jnp.repeat(jnp.arange(B, dtype=jnp.int32), N)
    flat = ((bidx * X + coords[:, 0]) * Y + coords[:, 1]) * Z + coords[:, 2]
    grid_ext = jnp.full((GRID + 1,), -1, dtype=jnp.int32).at[flat].set(
        jnp.arange(BN, dtype=jnp.int32))
    feats_pad = jnp.concatenate(
        [feats, jnp.zeros((NPAD, C), feats.dtype)], axis=0)
    w128 = jnp.concatenate([W, jnp.zeros((K, C, C), W.dtype)], axis=2)
    h = _h_matmul(feats_pad, w128)
    flat_pad = jnp.concatenate(
        [flat, jnp.zeros((PTOT + 2 * CH - BN,), jnp.int32)], axis=0)
    out_pre = _sc_conv(flat_pad, grid_ext, h)
    s, q = _bn_reduce(out_pre)
    gamma8 = jnp.broadcast_to(gamma.reshape(1, C), (8, C))
    beta8 = jnp.broadcast_to(beta.reshape(1, C), (8, C))
    y = _bn_norm(out_pre, s, q, gamma8, beta8)
    return y.reshape(B, N, C)


# speedup vs baseline: 1.2300x; 1.0530x over previous
import functools

import jax
import jax.numpy as jnp
from jax import lax
from jax.experimental import pallas as pl
from jax.experimental.pallas import tpu as pltpu
from jax.experimental.pallas import tpu_sc as plsc

B, N, C = 2, 25000, 64
BN = B * N
X = Y = Z = 128
GRID = B * X * Y * Z
K = 27
NPAD = 1200
NP = BN + NPAD
PADMASK = 1023

NW = 32
CH = 128
NCHUNK = 13
PW = CH * NCHUNK
PTOT = NW * PW

_OFFS = [(dx, dy, dz) for dx in (-1, 0, 1) for dy in (-1, 0, 1) for dz in (-1, 0, 1)]

MXBLK = 10240
RBLK = 1000


def _h_matmul_body(f_ref, w_ref, o_ref):
    o_ref[...] = jnp.dot(f_ref[...], w_ref[0],
                         preferred_element_type=jnp.float32)


def _h_matmul(feats_pad, w):
    nblk = NP // MXBLK
    return pl.pallas_call(
        _h_matmul_body,
        grid=(nblk, K),
        in_specs=[
            pl.BlockSpec((MXBLK, C), lambda j, k: (j, 0)),
            pl.BlockSpec((1, C, 2 * C), lambda j, k: (k, 0, 0)),
        ],
        out_specs=pl.BlockSpec((MXBLK, 2 * C), lambda j, k: (k * nblk + j, 0)),
        out_shape=jax.ShapeDtypeStruct((K * NP, 2 * C), jnp.float32),
    )(feats_pad, w)


_sc_mesh = plsc.VectorSubcoreMesh(core_axis_name="c", subcore_axis_name="s")


@functools.partial(
    pl.kernel,
    out_type=jax.ShapeDtypeStruct((PTOT, 2 * C), jnp.float32),
    mesh=_sc_mesh,
    scratch_types=[
        pltpu.VMEM((2, CH), jnp.int32),
        pltpu.VMEM((K * CH,), jnp.int32),
        pltpu.VMEM((K * CH,), jnp.int32),
        pltpu.VMEM((2 * K, CH), jnp.int32),
        pltpu.VMEM((2 * CH, 2 * C), jnp.float32),
        pltpu.SemaphoreType.DMA,
        pltpu.SemaphoreType.DMA,
        pltpu.SemaphoreType.DMA,
        pltpu.SemaphoreType.DMA,
        pltpu.SemaphoreType.DMA,
        pltpu.SemaphoreType.DMA,
        pltpu.SemaphoreType.DMA,
    ],
)
def _sc_conv(flat_hbm, grid_hbm, h_hbm, out_hbm,
             flat_v, nf, gv, ridx, acc, sem_f0, sem_f1, sem_g,
             sem_h0, sem_h1, sem_o0, sem_o1):
    wid = lax.axis_index("s") * 2 + lax.axis_index("c")
    zero16 = jnp.zeros((16,), jnp.float32)
    sem_f = [sem_f0, sem_f1]
    sem_h = [sem_h0, sem_h1]
    sem_o = [sem_o0, sem_o1]

    def prep(ci, par, out_pending):
        base = wid * PW + ci * CH
        pltpu.make_async_copy(flat_hbm.at[pl.ds(base, CH)],
                              flat_v.at[par], sem_f[par]).wait()

        def nf_body(g, carry):
            f = flat_v[par, pl.ds(g * 16, 16)]
            zc = f & 127
            yc = (f >> 7) & 127
            xc = (f >> 14) & 127
            for k, (dx, dy, dz) in enumerate(_OFFS):
                conds = []
                if dx == -1: conds.append(xc >= 1)
                if dx == 1: conds.append(xc <= X - 2)
                if dy == -1: conds.append(yc >= 1)
                if dy == 1: conds.append(yc <= Y - 2)
                if dz == -1: conds.append(zc >= 1)
                if dz == 1: conds.append(zc <= Z - 2)
                nfv = f + (dx * (Y * Z) + dy * Z + dz)
                if conds:
                    m = conds[0]
                    for c2 in conds[1:]:
                        m = m & c2
                    nfv = jnp.where(m, nfv, GRID)
                nf[pl.ds(k * CH + g * 16, 16)] = nfv
            return carry

        lax.fori_loop(0, CH // 16, nf_body, 0)
        pltpu.async_copy(flat_hbm.at[pl.ds(base + 2 * CH, CH)],
                         flat_v.at[par], sem_f[par])
        pltpu.async_copy(grid_hbm.at[nf], gv, sem_g).wait()

        def fix_body(g, carry):
            pvec = base + g * 16 + lax.iota(jnp.int32, 16)
            padv = BN + (pvec & PADMASK)
            for k in range(K):
                gval = gv[pl.ds(k * CH + g * 16, 16)]
                ridx[par * K + k, pl.ds(g * 16, 16)] = (
                    jnp.where(gval >= 0, gval, padv) + k * NP)
            return carry

        lax.fori_loop(0, CH // 16, fix_body, 0)
        if out_pending:
            pltpu.make_async_copy(out_hbm.at[pl.ds(0, CH)],
                                  acc.at[pl.ds(par * CH, CH)], sem_o[par]).wait()

        def zero_body(r, carry):
            for cb in range(2 * C // 16):
                acc[par * CH + r, pl.ds(cb * 16, 16)] = zero16
            return carry

        lax.fori_loop(0, CH, zero_body, 0)
        for k in range(K):
            pltpu.async_copy(h_hbm.at[ridx.at[par * K + k]],
                             acc.at[pl.ds(par * CH, CH)], sem_h[par], add=True)
        return base

    def fin(ci, par):
        base = wid * PW + ci * CH
        for _ in range(K):
            pltpu.make_async_copy(h_hbm.at[pl.ds(0, CH)],
                                  acc.at[pl.ds(par * CH, CH)], sem_h[par]).wait()
        pltpu.async_copy(acc.at[pl.ds(par * CH, CH)],
                         out_hbm.at[pl.ds(base, CH)], sem_o[par])

    pltpu.async_copy(flat_hbm.at[pl.ds(wid * PW, CH)], flat_v.at[0], sem_f0)
    pltpu.async_copy(flat_hbm.at[pl.ds(wid * PW + CH, CH)], flat_v.at[1], sem_f1)
    prep(0, 0, False)

    def loop_body(i, carry):
        c0 = 2 * i

        @pl.when(i >= 1)
        def _():
            pltpu.make_async_copy(out_hbm.at[pl.ds(0, CH)],
                                  acc.at[pl.ds(CH, CH)], sem_o1).wait()
        prep(c0 + 1, 1, False)
        fin(c0, 0)
        prep(c0 + 2, 0, True)
        fin(c0 + 1, 1)
        return carry

    lax.fori_loop(0, (NCHUNK - 1) // 2, loop_body, 0)
    fin(NCHUNK - 1, 0)
    pltpu.make_async_copy(out_hbm.at[pl.ds(0, CH)],
                          acc.at[pl.ds(0, CH)], sem_o0).wait()
    pltpu.make_async_copy(out_hbm.at[pl.ds(0, CH)],
                          acc.at[pl.ds(CH, CH)], sem_o1).wait()
    pltpu.make_async_copy(flat_hbm.at[pl.ds(0, CH)],
                          flat_v.at[0], sem_f0).wait()
    pltpu.make_async_copy(flat_hbm.at[pl.ds(0, CH)],
                          flat_v.at[1], sem_f1).wait()


def _bn_reduce_body(x_ref, s_ref, q_ref):
    i = pl.program_id(0)
    xs = x_ref[:, 0:C].astype(jnp.float32)
    s = jnp.broadcast_to(jnp.sum(xs, axis=0, keepdims=True), (8, C))
    q = jnp.broadcast_to(jnp.sum(xs * xs, axis=0, keepdims=True), (8, C))

    @pl.when(i == 0)
    def _():
        s_ref[...] = s
        q_ref[...] = q

    @pl.when(i > 0)
    def _():
        s_ref[...] += s
        q_ref[...] += q


def _bn_reduce(out_pre):
    return pl.pallas_call(
        _bn_reduce_body,
        grid=(BN // RBLK,),
        in_specs=[pl.BlockSpec((RBLK, 2 * C), lambda i: (i, 0))],
        out_specs=[pl.BlockSpec((8, C), lambda i: (0, 0))] * 2,
        out_shape=[jax.ShapeDtypeStruct((8, C), jnp.float32)] * 2,
    )(out_pre)


def _bn_norm_body(x_ref, s_ref, q_ref, g_ref, b_ref, o_ref):
    mean = s_ref[0:1, :] * (1.0 / BN)
    var = q_ref[0:1, :] * (1.0 / BN) - mean * mean
    inv = lax.rsqrt(var + 1e-5)
    scale = g_ref[0:1, :] * inv
    shift = b_ref[0:1, :] - mean * scale
    o_ref[...] = jnp.maximum(
        x_ref[:, 0:C].astype(jnp.float32) * scale + shift, 0.0)


def _bn_norm(out_pre, s, q, gamma8, beta8):
    return pl.pallas_call(
        _bn_norm_body,
        grid=(BN // RBLK,),
        in_specs=[
            pl.BlockSpec((RBLK, 2 * C), lambda i: (i, 0)),
            pl.BlockSpec((8, C), lambda i: (0, 0)),
            pl.BlockSpec((8, C), lambda i: (0, 0)),
            pl.BlockSpec((8, C), lambda i: (0, 0)),
            pl.BlockSpec((8, C), lambda i: (0, 0)),
        ],
        out_specs=pl.BlockSpec((RBLK, C), lambda i: (i, 0)),
        out_shape=jax.ShapeDtypeStruct((BN, C), jnp.float32),
    )(out_pre, s, q, gamma8, beta8)


def kernel(features, coordinates, spatial_shape, batch_size, W, gamma, beta):
    feats = features.reshape(BN, C)
    coords = coordinates.reshape(BN, 3).astype(jnp.int32)
    bidx = jnp.repeat(jnp.arange(B, dtype=jnp.int32), N)
    flat = ((bidx * X + coords[:, 0]) * Y + coords[:, 1]) * Z + coords[:, 2]
    grid_ext = jnp.full((GRID + 1,), -1, dtype=jnp.int32).at[flat].set(
        jnp.arange(BN, dtype=jnp.int32))
    feats_pad = jnp.concatenate(
        [feats, jnp.zeros((NPAD, C), feats.dtype)], axis=0)
    w128 = jnp.concatenate([W, jnp.zeros((K, C, C), W.dtype)], axis=2)
    h = _h_matmul(feats_pad, w128)
    flat_pad = jnp.concatenate(
        [flat, jnp.zeros((PTOT + 2 * CH - BN,), jnp.int32)], axis=0)
    out_pre = _sc_conv(flat_pad, grid_ext, h)
    s, q = _bn_reduce(out_pre)
    gamma8 = jnp.broadcast_to(gamma.reshape(1, C), (8, C))
    beta8 = jnp.broadcast_to(beta.reshape(1, C), (8, C))
    y = _bn_norm(out_pre, s, q, gamma8, beta8)
    return y.reshape(B, N, C)


# speedup vs baseline: 1.2415x; 1.0094x over previous
import functools

import jax
import jax.numpy as jnp
from jax import lax
from jax.experimental import pallas as pl
from jax.experimental.pallas import tpu as pltpu
from jax.experimental.pallas import tpu_sc as plsc

B, N, C = 2, 25000, 64
BN = B * N
X = Y = Z = 128
GRID = B * X * Y * Z
K = 27
NPAD = 1200
NP = BN + NPAD
PADMASK = 1023

NW = 32
CH = 128
NCHUNK = 13
PW = CH * NCHUNK
PTOT = NW * PW

_OFFS = [(dx, dy, dz) for dx in (-1, 0, 1) for dy in (-1, 0, 1) for dz in (-1, 0, 1)]

MXBLK = 25600
RBLK = 1000


def _h_matmul_body(f_ref, w_ref, o_ref):
    o_ref[...] = jnp.dot(f_ref[...], w_ref[0],
                         preferred_element_type=jnp.float32)


def _h_matmul(feats_pad, w):
    nblk = NP // MXBLK
    return pl.pallas_call(
        _h_matmul_body,
        grid=(nblk, K),
        in_specs=[
            pl.BlockSpec((MXBLK, C), lambda j, k: (j, 0)),
            pl.BlockSpec((1, C, 2 * C), lambda j, k: (k, 0, 0)),
        ],
        out_specs=pl.BlockSpec((MXBLK, 2 * C), lambda j, k: (k * nblk + j, 0)),
        out_shape=jax.ShapeDtypeStruct((K * NP, 2 * C), jnp.float32),
    )(feats_pad, w)


_sc_mesh = plsc.VectorSubcoreMesh(core_axis_name="c", subcore_axis_name="s")


@functools.partial(
    pl.kernel,
    out_type=jax.ShapeDtypeStruct((PTOT, 2 * C), jnp.float32),
    mesh=_sc_mesh,
    scratch_types=[
        pltpu.VMEM((2, CH), jnp.int32),
        pltpu.VMEM((K * CH,), jnp.int32),
        pltpu.VMEM((K * CH,), jnp.int32),
        pltpu.VMEM((2 * K, CH), jnp.int32),
        pltpu.VMEM((2 * CH, 2 * C), jnp.float32),
        pltpu.SemaphoreType.DMA,
        pltpu.SemaphoreType.DMA,
        pltpu.SemaphoreType.DMA,
        pltpu.SemaphoreType.DMA,
        pltpu.SemaphoreType.DMA,
        pltpu.SemaphoreType.DMA,
        pltpu.SemaphoreType.DMA,
    ],
)
def _sc_conv(flat_hbm, grid_hbm, h_hbm, out_hbm,
             flat_v, nf, gv, ridx, acc, sem_f0, sem_f1, sem_g,
             sem_h0, sem_h1, sem_o0, sem_o1):
    wid = lax.axis_index("s") * 2 + lax.axis_index("c")
    zero16 = jnp.zeros((16,), jnp.float32)
    sem_f = [sem_f0, sem_f1]
    sem_h = [sem_h0, sem_h1]
    sem_o = [sem_o0, sem_o1]

    def prep(ci, par, out_pending):
        base = wid * PW + ci * CH
        pltpu.make_async_copy(flat_hbm.at[pl.ds(base, CH)],
                              flat_v.at[par], sem_f[par]).wait()

        def nf_body(g, carry):
            f = flat_v[par, pl.ds(g * 16, 16)]
            zc = f & 127
            yc = (f >> 7) & 127
            xc = (f >> 14) & 127
            for k, (dx, dy, dz) in enumerate(_OFFS):
                conds = []
                if dx == -1: conds.append(xc >= 1)
                if dx == 1: conds.append(xc <= X - 2)
                if dy == -1: conds.append(yc >= 1)
                if dy == 1: conds.append(yc <= Y - 2)
                if dz == -1: conds.append(zc >= 1)
                if dz == 1: conds.append(zc <= Z - 2)
                nfv = f + (dx * (Y * Z) + dy * Z + dz)
                if conds:
                    m = conds[0]
                    for c2 in conds[1:]:
                        m = m & c2
                    nfv = jnp.where(m, nfv, GRID)
                nf[pl.ds(k * CH + g * 16, 16)] = nfv
            return carry

        lax.fori_loop(0, CH // 16, nf_body, 0)
        pltpu.async_copy(flat_hbm.at[pl.ds(base + 2 * CH, CH)],
                         flat_v.at[par], sem_f[par])
        pltpu.async_copy(grid_hbm.at[nf], gv, sem_g).wait()

        def fix_body(g, carry):
            pvec = base + g * 16 + lax.iota(jnp.int32, 16)
            padv = BN + (pvec & PADMASK)
            for k in range(K):
                gval = gv[pl.ds(k * CH + g * 16, 16)]
                ridx[par * K + k, pl.ds(g * 16, 16)] = (
                    jnp.where(gval >= 0, gval, padv) + k * NP)
            return carry

        lax.fori_loop(0, CH // 16, fix_body, 0)
        if out_pending:
            pltpu.make_async_copy(out_hbm.at[pl.ds(0, CH)],
                                  acc.at[pl.ds(par * CH, CH)], sem_o[par]).wait()

        def zero_body(r, carry):
            for cb in range(2 * C // 16):
                acc[par * CH + r, pl.ds(cb * 16, 16)] = zero16
            return carry

        lax.fori_loop(0, CH, zero_body, 0)
        for k in range(K):
            pltpu.async_copy(h_hbm.at[ridx.at[par * K + k]],
                             acc.at[pl.ds(par * CH, CH)], sem_h[par], add=True)
        return base

    def fin(ci, par):
        base = wid * PW + ci * CH
        for _ in range(K):
            pltpu.make_async_copy(h_hbm.at[pl.ds(0, CH)],
                                  acc.at[pl.ds(par * CH, CH)], sem_h[par]).wait()
        pltpu.async_copy(acc.at[pl.ds(par * CH, CH)],
                         out_hbm.at[pl.ds(base, CH)], sem_o[par])

    pltpu.async_copy(flat_hbm.at[pl.ds(wid * PW, CH)], flat_v.at[0], sem_f0)
    pltpu.async_copy(flat_hbm.at[pl.ds(wid * PW + CH, CH)], flat_v.at[1], sem_f1)
    prep(0, 0, False)

    def loop_body(i, carry):
        c0 = 2 * i

        @pl.when(i >= 1)
        def _():
            pltpu.make_async_copy(out_hbm.at[pl.ds(0, CH)],
                                  acc.at[pl.ds(CH, CH)], sem_o1).wait()
        prep(c0 + 1, 1, False)
        fin(c0, 0)
        prep(c0 + 2, 0, True)
        fin(c0 + 1, 1)
        return carry

    lax.fori_loop(0, (NCHUNK - 1) // 2, loop_body, 0)
    fin(NCHUNK - 1, 0)
    pltpu.make_async_copy(out_hbm.at[pl.ds(0, CH)],
                          acc.at[pl.ds(0, CH)], sem_o0).wait()
    pltpu.make_async_copy(out_hbm.at[pl.ds(0, CH)],
                          acc.at[pl.ds(CH, CH)], sem_o1).wait()
    pltpu.make_async_copy(flat_hbm.at[pl.ds(0, CH)],
                          flat_v.at[0], sem_f0).wait()
    pltpu.make_async_copy(flat_hbm.at[pl.ds(0, CH)],
                          flat_v.at[1], sem_f1).wait()


def _bn_reduce_body(x_ref, s_ref, q_ref):
    i = pl.program_id(0)
    xs = x_ref[:, 0:C].astype(jnp.float32)
    s = jnp.broadcast_to(jnp.sum(xs, axis=0, keepdims=True), (8, C))
    q = jnp.broadcast_to(jnp.sum(xs * xs, axis=0, keepdims=True), (8, C))

    @pl.when(i == 0)
    def _():
        s_ref[...] = s
        q_ref[...] = q

    @pl.when(i > 0)
    def _():
        s_ref[...] += s
        q_ref[...] += q


def _bn_reduce(out_pre):
    return pl.pallas_call(
        _bn_reduce_body,
        grid=(BN // RBLK,),
        in_specs=[pl.BlockSpec((RBLK, 2 * C), lambda i: (i, 0))],
        out_specs=[pl.BlockSpec((8, C), lambda i: (0, 0))] * 2,
        out_shape=[jax.ShapeDtypeStruct((8, C), jnp.float32)] * 2,
    )(out_pre)


def _bn_norm_body(x_ref, s_ref, q_ref, g_ref, b_ref, o_ref):
    mean = s_ref[0:1, :] * (1.0 / BN)
    var = q_ref[0:1, :] * (1.0 / BN) - mean * mean
    inv = lax.rsqrt(var + 1e-5)
    scale = g_ref[0:1, :] * inv
    shift = b_ref[0:1, :] - mean * scale
    o_ref[...] = jnp.maximum(
        x_ref[:, 0:C].astype(jnp.float32) * scale + shift, 0.0)


def _bn_norm(out_pre, s, q, gamma8, beta8):
    return pl.pallas_call(
        _bn_norm_body,
        grid=(BN // RBLK,),
        in_specs=[
            pl.BlockSpec((RBLK, 2 * C), lambda i: (i, 0)),
            pl.BlockSpec((8, C), lambda i: (0, 0)),
            pl.BlockSpec((8, C), lambda i: (0, 0)),
            pl.BlockSpec((8, C), lambda i: (0, 0)),
            pl.BlockSpec((8, C), lambda i: (0, 0)),
        ],
        out_specs=pl.BlockSpec((RBLK, C), lambda i: (i, 0)),
        out_shape=jax.ShapeDtypeStruct((BN, C), jnp.float32),
    )(out_pre, s, q, gamma8, beta8)


def kernel(features, coordinates, spatial_shape, batch_size, W, gamma, beta):
    feats = features.reshape(BN, C)
    coords = coordinates.reshape(BN, 3).astype(jnp.int32)
    bidx = jnp.repeat(jnp.arange(B, dtype=jnp.int32), N)
    flat = ((bidx * X + coords[:, 0]) * Y + coords[:, 1]) * Z + coords[:, 2]
    grid_ext = jnp.full((GRID + 1,), -1, dtype=jnp.int32).at[flat].set(
        jnp.arange(BN, dtype=jnp.int32))
    feats_pad = jnp.concatenate(
        [feats, jnp.zeros((NPAD, C), feats.dtype)], axis=0)
    w128 = jnp.concatenate([W, jnp.zeros((K, C, C), W.dtype)], axis=2)
    h = _h_matmul(feats_pad, w128)
    flat_pad = jnp.concatenate(
        [flat, jnp.zeros((PTOT + 2 * CH - BN,), jnp.int32)], axis=0)
    out_pre = _sc_conv(flat_pad, grid_ext, h)
    s, q = _bn_reduce(out_pre)
    gamma8 = jnp.broadcast_to(gamma.reshape(1, C), (8, C))
    beta8 = jnp.broadcast_to(beta.reshape(1, C), (8, C))
    y = _bn_norm(out_pre, s, q, gamma8, beta8)
    return y.reshape(B, N, C)


# speedup vs baseline: 1.2817x; 1.0324x over previous
import functools

import jax
import jax.numpy as jnp
from jax import lax
from jax.experimental import pallas as pl
from jax.experimental.pallas import tpu as pltpu
from jax.experimental.pallas import tpu_sc as plsc

B, N, C = 2, 25000, 64
BN = B * N
X = Y = Z = 128
GRID = B * X * Y * Z
K = 27
NPAD = 1200
NP = BN + NPAD
PADMASK = 1023

NW = 32
CH = 128
NCHUNK = 13
PW = CH * NCHUNK
PTOT = NW * PW

_OFFS = [(dx, dy, dz) for dx in (-1, 0, 1) for dy in (-1, 0, 1) for dz in (-1, 0, 1)]

MXBLK = 25600
RBLK = 10000


def _h_matmul_body(f_ref, w_ref, o_ref):
    o_ref[...] = jnp.dot(f_ref[...], w_ref[0],
                         preferred_element_type=jnp.float32)


def _h_matmul(feats_pad, w):
    nblk = NP // MXBLK
    return pl.pallas_call(
        _h_matmul_body,
        grid=(nblk, K),
        in_specs=[
            pl.BlockSpec((MXBLK, C), lambda j, k: (j, 0)),
            pl.BlockSpec((1, C, 2 * C), lambda j, k: (k, 0, 0)),
        ],
        out_specs=pl.BlockSpec((MXBLK, 2 * C), lambda j, k: (k * nblk + j, 0)),
        out_shape=jax.ShapeDtypeStruct((K * NP, 2 * C), jnp.float32),
    )(feats_pad, w)


_sc_mesh = plsc.VectorSubcoreMesh(core_axis_name="c", subcore_axis_name="s")


@functools.partial(
    pl.kernel,
    out_type=jax.ShapeDtypeStruct((PTOT, 2 * C), jnp.float32),
    mesh=_sc_mesh,
    scratch_types=[
        pltpu.VMEM((2, CH), jnp.int32),
        pltpu.VMEM((K * CH,), jnp.int32),
        pltpu.VMEM((K * CH,), jnp.int32),
        pltpu.VMEM((2 * K, CH), jnp.int32),
        pltpu.VMEM((2 * CH, 2 * C), jnp.float32),
        pltpu.SemaphoreType.DMA,
        pltpu.SemaphoreType.DMA,
        pltpu.SemaphoreType.DMA,
        pltpu.SemaphoreType.DMA,
        pltpu.SemaphoreType.DMA,
        pltpu.SemaphoreType.DMA,
        pltpu.SemaphoreType.DMA,
    ],
)
def _sc_conv(flat_hbm, grid_hbm, h_hbm, out_hbm,
             flat_v, nf, gv, ridx, acc, sem_f0, sem_f1, sem_g,
             sem_h0, sem_h1, sem_o0, sem_o1):
    wid = lax.axis_index("s") * 2 + lax.axis_index("c")
    zero16 = jnp.zeros((16,), jnp.float32)
    sem_f = [sem_f0, sem_f1]
    sem_h = [sem_h0, sem_h1]
    sem_o = [sem_o0, sem_o1]

    def prep(ci, par, out_pending):
        base = wid * PW + ci * CH
        pltpu.make_async_copy(flat_hbm.at[pl.ds(base, CH)],
                              flat_v.at[par], sem_f[par]).wait()

        def nf_body(g, carry):
            f = flat_v[par, pl.ds(g * 16, 16)]
            zc = f & 127
            yc = (f >> 7) & 127
            xc = (f >> 14) & 127
            for k, (dx, dy, dz) in enumerate(_OFFS):
                conds = []
                if dx == -1: conds.append(xc >= 1)
                if dx == 1: conds.append(xc <= X - 2)
                if dy == -1: conds.append(yc >= 1)
                if dy == 1: conds.append(yc <= Y - 2)
                if dz == -1: conds.append(zc >= 1)
                if dz == 1: conds.append(zc <= Z - 2)
                nfv = f + (dx * (Y * Z) + dy * Z + dz)
                if conds:
                    m = conds[0]
                    for c2 in conds[1:]:
                        m = m & c2
                    nfv = jnp.where(m, nfv, GRID)
                nf[pl.ds(k * CH + g * 16, 16)] = nfv
            return carry

        lax.fori_loop(0, CH // 16, nf_body, 0)
        pltpu.async_copy(flat_hbm.at[pl.ds(base + 2 * CH, CH)],
                         flat_v.at[par], sem_f[par])
        pltpu.async_copy(grid_hbm.at[nf], gv, sem_g).wait()

        def fix_body(g, carry):
            pvec = base + g * 16 + lax.iota(jnp.int32, 16)
            padv = BN + (pvec & PADMASK)
            for k in range(K):
                gval = gv[pl.ds(k * CH + g * 16, 16)]
                ridx[par * K + k, pl.ds(g * 16, 16)] = (
                    jnp.where(gval >= 0, gval, padv) + k * NP)
            return carry

        lax.fori_loop(0, CH // 16, fix_body, 0)
        if out_pending:
            pltpu.make_async_copy(out_hbm.at[pl.ds(0, CH)],
                                  acc.at[pl.ds(par * CH, CH)], sem_o[par]).wait()

        def zero_body(r, carry):
            for cb in range(2 * C // 16):
                acc[par * CH + r, pl.ds(cb * 16, 16)] = zero16
            return carry

        lax.fori_loop(0, CH, zero_body, 0)
        for k in range(K):
            pltpu.async_copy(h_hbm.at[ridx.at[par * K + k]],
                             acc.at[pl.ds(par * CH, CH)], sem_h[par], add=True)
        return base

    def fin(ci, par):
        base = wid * PW + ci * CH
        for _ in range(K):
            pltpu.make_async_copy(h_hbm.at[pl.ds(0, CH)],
                                  acc.at[pl.ds(par * CH, CH)], sem_h[par]).wait()
        pltpu.async_copy(acc.at[pl.ds(par * CH, CH)],
                         out_hbm.at[pl.ds(base, CH)], sem_o[par])

    pltpu.async_copy(flat_hbm.at[pl.ds(wid * PW, CH)], flat_v.at[0], sem_f0)
    pltpu.async_copy(flat_hbm.at[pl.ds(wid * PW + CH, CH)], flat_v.at[1], sem_f1)
    prep(0, 0, False)

    def loop_body(i, carry):
        c0 = 2 * i

        @pl.when(i >= 1)
        def _():
            pltpu.make_async_copy(out_hbm.at[pl.ds(0, CH)],
                                  acc.at[pl.ds(CH, CH)], sem_o1).wait()
        prep(c0 + 1, 1, False)
        fin(c0, 0)
        prep(c0 + 2, 0, True)
        fin(c0 + 1, 1)
        return carry

    lax.fori_loop(0, (NCHUNK - 1) // 2, loop_body, 0)
    fin(NCHUNK - 1, 0)
    pltpu.make_async_copy(out_hbm.at[pl.ds(0, CH)],
                          acc.at[pl.ds(0, CH)], sem_o0).wait()
    pltpu.make_async_copy(out_hbm.at[pl.ds(0, CH)],
                          acc.at[pl.ds(CH, CH)], sem_o1).wait()
    pltpu.make_async_copy(flat_hbm.at[pl.ds(0, CH)],
                          flat_v.at[0], sem_f0).wait()
    pltpu.make_async_copy(flat_hbm.at[pl.ds(0, CH)],
                          flat_v.at[1], sem_f1).wait()


def _bn_reduce_body(x_ref, s_ref, q_ref):
    i = pl.program_id(0)
    xs = x_ref[:, 0:C].astype(jnp.float32)
    s = jnp.broadcast_to(jnp.sum(xs, axis=0, keepdims=True), (8, C))
    q = jnp.broadcast_to(jnp.sum(xs * xs, axis=0, keepdims=True), (8, C))

    @pl.when(i == 0)
    def _():
        s_ref[...] = s
        q_ref[...] = q

    @pl.when(i > 0)
    def _():
        s_ref[...] += s
        q_ref[...] += q


def _bn_reduce(out_pre):
    return pl.pallas_call(
        _bn_reduce_body,
        grid=(BN // RBLK,),
        in_specs=[pl.BlockSpec((RBLK, 2 * C), lambda i: (i, 0))],
        out_specs=[pl.BlockSpec((8, C), lambda i: (0, 0))] * 2,
        out_shape=[jax.ShapeDtypeStruct((8, C), jnp.float32)] * 2,
    )(out_pre)


def _bn_norm_body(x_ref, s_ref, q_ref, g_ref, b_ref, o_ref):
    mean = s_ref[0:1, :] * (1.0 / BN)
    var = q_ref[0:1, :] * (1.0 / BN) - mean * mean
    inv = lax.rsqrt(var + 1e-5)
    scale = g_ref[0:1, :] * inv
    shift = b_ref[0:1, :] - mean * scale
    o_ref[...] = jnp.maximum(
        x_ref[:, 0:C].astype(jnp.float32) * scale + shift, 0.0)


def _bn_norm(out_pre, s, q, gamma8, beta8):
    return pl.pallas_call(
        _bn_norm_body,
        grid=(BN // RBLK,),
        in_specs=[
            pl.BlockSpec((RBLK, 2 * C), lambda i: (i, 0)),
            pl.BlockSpec((8, C), lambda i: (0, 0)),
            pl.BlockSpec((8, C), lambda i: (0, 0)),
            pl.BlockSpec((8, C), lambda i: (0, 0)),
            pl.BlockSpec((8, C), lambda i: (0, 0)),
        ],
        out_specs=pl.BlockSpec((RBLK, C), lambda i: (i, 0)),
        out_shape=jax.ShapeDtypeStruct((BN, C), jnp.float32),
    )(out_pre, s, q, gamma8, beta8)


def kernel(features, coordinates, spatial_shape, batch_size, W, gamma, beta):
    feats = features.reshape(BN, C)
    coords = coordinates.reshape(BN, 3).astype(jnp.int32)
    bidx = jnp.repeat(jnp.arange(B, dtype=jnp.int32), N)
    flat = ((bidx * X + coords[:, 0]) * Y + coords[:, 1]) * Z + coords[:, 2]
    grid_ext = jnp.full((GRID + 1,), -1, dtype=jnp.int32).at[flat].set(
        jnp.arange(BN, dtype=jnp.int32))
    feats_pad = jnp.concatenate(
        [feats, jnp.zeros((NPAD, C), feats.dtype)], axis=0)
    w128 = jnp.concatenate([W, jnp.zeros((K, C, C), W.dtype)], axis=2)
    h = _h_matmul(feats_pad, w128)
    flat_pad = jnp.concatenate(
        [flat, jnp.zeros((PTOT + 2 * CH - BN,), jnp.int32)], axis=0)
    out_pre = _sc_conv(flat_pad, grid_ext, h)
    s, q = _bn_reduce(out_pre)
    gamma8 = jnp.broadcast_to(gamma.reshape(1, C), (8, C))
    beta8 = jnp.broadcast_to(beta.reshape(1, C), (8, C))
    y = _bn_norm(out_pre, s, q, gamma8, beta8)
    return y.reshape(B, N, C)


# speedup vs baseline: 1.4421x; 1.1251x over previous
import functools

import jax
import jax.numpy as jnp
from jax import lax
from jax.experimental import pallas as pl
from jax.experimental.pallas import tpu as pltpu
from jax.experimental.pallas import tpu_sc as plsc

B, N, C = 2, 25000, 64
BN = B * N
X = Y = Z = 128
GRID = B * X * Y * Z
K = 27
NPAD = 1200
NP = BN + NPAD
PADMASK = 1023

NW = 32
CH = 128
NCHUNK = 13
PW = CH * NCHUNK
PTOT = NW * PW

_OFFS = [(dx, dy, dz) for dx in (-1, 0, 1) for dy in (-1, 0, 1) for dz in (-1, 0, 1)]

MXBLK = 25600
RBLK = 10000


def _h_matmul_body(f_ref, w_ref, o_ref):
    o_ref[...] = jnp.dot(f_ref[...], w_ref[0],
                         preferred_element_type=jnp.float32)


def _h_matmul(feats_pad, w):
    nblk = NP // MXBLK
    return pl.pallas_call(
        _h_matmul_body,
        grid=(nblk, K),
        in_specs=[
            pl.BlockSpec((MXBLK, C), lambda j, k: (j, 0)),
            pl.BlockSpec((1, C, 2 * C), lambda j, k: (k, 0, 0)),
        ],
        out_specs=pl.BlockSpec((MXBLK, 2 * C), lambda j, k: (k * nblk + j, 0)),
        out_shape=jax.ShapeDtypeStruct((K * NP, 2 * C), jnp.float32),
    )(feats_pad, w)


_sc_mesh = plsc.VectorSubcoreMesh(core_axis_name="c", subcore_axis_name="s")


@functools.partial(
    pl.kernel,
    out_type=jax.ShapeDtypeStruct((PTOT, 2 * C), jnp.float32),
    mesh=_sc_mesh,
    scratch_types=[
        pltpu.VMEM((2, CH), jnp.int32),
        pltpu.VMEM((K * CH,), jnp.int32),
        pltpu.VMEM((K * CH,), jnp.int32),
        pltpu.VMEM((2 * K, CH), jnp.int32),
        pltpu.VMEM((2 * CH, 2 * C), jnp.float32),
        pltpu.SemaphoreType.DMA,
        pltpu.SemaphoreType.DMA,
        pltpu.SemaphoreType.DMA,
        pltpu.SemaphoreType.DMA,
        pltpu.SemaphoreType.DMA,
        pltpu.SemaphoreType.DMA,
        pltpu.SemaphoreType.DMA,
    ],
)
def _sc_conv(flat_hbm, grid_hbm, h_hbm, out_hbm,
             flat_v, nf, gv, ridx, acc, sem_f0, sem_f1, sem_g,
             sem_h0, sem_h1, sem_o0, sem_o1):
    wid = lax.axis_index("s") * 2 + lax.axis_index("c")
    zero16 = jnp.zeros((16,), jnp.float32)
    sem_f = [sem_f0, sem_f1]
    sem_h = [sem_h0, sem_h1]
    sem_o = [sem_o0, sem_o1]

    def prep(ci, par, out_pending):
        base = wid * PW + ci * CH
        pltpu.make_async_copy(flat_hbm.at[pl.ds(base, CH)],
                              flat_v.at[par], sem_f[par]).wait()

        def nf_body(g, carry):
            f = flat_v[par, pl.ds(g * 16, 16)]
            zc = f & 127
            yc = (f >> 7) & 127
            xc = (f >> 14) & 127
            for k, (dx, dy, dz) in enumerate(_OFFS):
                conds = []
                if dx == -1: conds.append(xc >= 1)
                if dx == 1: conds.append(xc <= X - 2)
                if dy == -1: conds.append(yc >= 1)
                if dy == 1: conds.append(yc <= Y - 2)
                if dz == -1: conds.append(zc >= 1)
                if dz == 1: conds.append(zc <= Z - 2)
                nfv = f + (dx * (Y * Z) + dy * Z + dz)
                if conds:
                    m = conds[0]
                    for c2 in conds[1:]:
                        m = m & c2
                    nfv = jnp.where(m, nfv, GRID)
                nf[pl.ds(k * CH + g * 16, 16)] = nfv
            return carry

        lax.fori_loop(0, CH // 16, nf_body, 0)
        pltpu.async_copy(flat_hbm.at[pl.ds(base + 2 * CH, CH)],
                         flat_v.at[par], sem_f[par])
        pltpu.async_copy(grid_hbm.at[nf], gv, sem_g).wait()

        def fix_body(g, carry):
            pvec = base + g * 16 + lax.iota(jnp.int32, 16)
            padv = BN + (pvec & PADMASK)
            for k in range(K):
                gval = gv[pl.ds(k * CH + g * 16, 16)]
                ridx[par * K + k, pl.ds(g * 16, 16)] = (
                    jnp.where(gval >= 0, gval, padv) + k * NP)
            return carry

        lax.fori_loop(0, CH // 16, fix_body, 0)
        if out_pending:
            pltpu.make_async_copy(out_hbm.at[pl.ds(0, CH)],
                                  acc.at[pl.ds(par * CH, CH)], sem_o[par]).wait()

        def zero_body(r, carry):
            for cb in range(2 * C // 16):
                acc[par * CH + r, pl.ds(cb * 16, 16)] = zero16
            return carry

        lax.fori_loop(0, CH, zero_body, 0)
        for k in range(K):
            pltpu.async_copy(h_hbm.at[ridx.at[par * K + k]],
                             acc.at[pl.ds(par * CH, CH)], sem_h[par], add=True)
        return base

    def fin(ci, par):
        base = wid * PW + ci * CH
        for _ in range(K):
            pltpu.make_async_copy(h_hbm.at[pl.ds(0, CH)],
                                  acc.at[pl.ds(par * CH, CH)], sem_h[par]).wait()
        pltpu.async_copy(acc.at[pl.ds(par * CH, CH)],
                         out_hbm.at[pl.ds(base, CH)], sem_o[par])

    pltpu.async_copy(flat_hbm.at[pl.ds(wid * PW, CH)], flat_v.at[0], sem_f0)
    pltpu.async_copy(flat_hbm.at[pl.ds(wid * PW + CH, CH)], flat_v.at[1], sem_f1)
    prep(0, 0, False)

    def loop_body(i, carry):
        c0 = 2 * i

        @pl.when(i >= 1)
        def _():
            pltpu.make_async_copy(out_hbm.at[pl.ds(0, CH)],
                                  acc.at[pl.ds(CH, CH)], sem_o1).wait()
        prep(c0 + 1, 1, False)
        fin(c0, 0)
        prep(c0 + 2, 0, True)
        fin(c0 + 1, 1)
        return carry

    lax.fori_loop(0, (NCHUNK - 1) // 2, loop_body, 0)
    fin(NCHUNK - 1, 0)
    pltpu.make_async_copy(out_hbm.at[pl.ds(0, CH)],
                          acc.at[pl.ds(0, CH)], sem_o0).wait()
    pltpu.make_async_copy(out_hbm.at[pl.ds(0, CH)],
                          acc.at[pl.ds(CH, CH)], sem_o1).wait()
    pltpu.make_async_copy(flat_hbm.at[pl.ds(0, CH)],
                          flat_v.at[0], sem_f0).wait()
    pltpu.make_async_copy(flat_hbm.at[pl.ds(0, CH)],
                          flat_v.at[1], sem_f1).wait()


def _bn_reduce_body(x_ref, s_ref, q_ref):
    i = pl.program_id(0)
    xs = x_ref[:, 0:C].astype(jnp.float32)
    s = jnp.broadcast_to(jnp.sum(xs, axis=0, keepdims=True), (8, C))
    q = jnp.broadcast_to(jnp.sum(xs * xs, axis=0, keepdims=True), (8, C))

    @pl.when(i == 0)
    def _():
        s_ref[...] = s
        q_ref[...] = q

    @pl.when(i > 0)
    def _():
        s_ref[...] += s
        q_ref[...] += q


def _bn_reduce(out_pre):
    return pl.pallas_call(
        _bn_reduce_body,
        grid=(BN // RBLK,),
        in_specs=[pl.BlockSpec((RBLK, 2 * C), lambda i: (i, 0))],
        out_specs=[pl.BlockSpec((8, C), lambda i: (0, 0))] * 2,
        out_shape=[jax.ShapeDtypeStruct((8, C), jnp.float32)] * 2,
    )(out_pre)


def _bn_norm_body(x_ref, s_ref, q_ref, g_ref, b_ref, o_ref):
    mean = s_ref[0:1, :] * (1.0 / BN)
    var = q_ref[0:1, :] * (1.0 / BN) - mean * mean
    inv = lax.rsqrt(var + 1e-5)
    scale = g_ref[0:1, :] * inv
    shift = b_ref[0:1, :] - mean * scale
    o_ref[...] = jnp.maximum(
        x_ref[:, 0:C].astype(jnp.float32) * scale + shift, 0.0)


def _bn_norm(out_pre, s, q, gamma8, beta8):
    return pl.pallas_call(
        _bn_norm_body,
        grid=(BN // RBLK,),
        in_specs=[
            pl.BlockSpec((RBLK, 2 * C), lambda i: (i, 0)),
            pl.BlockSpec((8, C), lambda i: (0, 0)),
            pl.BlockSpec((8, C), lambda i: (0, 0)),
            pl.BlockSpec((8, C), lambda i: (0, 0)),
            pl.BlockSpec((8, C), lambda i: (0, 0)),
        ],
        out_specs=pl.BlockSpec((RBLK, C), lambda i: (i, 0)),
        out_shape=jax.ShapeDtypeStruct((BN, C), jnp.float32),
    )(out_pre, s, q, gamma8, beta8)


def kernel(features, coordinates, spatial_shape, batch_size, W, gamma, beta):
    feats = features.reshape(BN, C)
    coords = coordinates.reshape(BN, 3).astype(jnp.int32)
    bidx = jnp.repeat(jnp.arange(B, dtype=jnp.int32), N)
    flat = ((bidx * X + coords[:, 0]) * Y + coords[:, 1]) * Z + coords[:, 2]
    grid_ext = jnp.full((GRID + 1,), -1, dtype=jnp.int32).at[flat].max(
        jnp.arange(BN, dtype=jnp.int32))
    feats_pad = jnp.concatenate(
        [feats, jnp.zeros((NPAD, C), feats.dtype)], axis=0)
    w128 = jnp.concatenate([W, jnp.zeros((K, C, C), W.dtype)], axis=2)
    h = _h_matmul(feats_pad, w128)
    flat_pad = jnp.concatenate(
        [flat, jnp.zeros((PTOT + 2 * CH - BN,), jnp.int32)], axis=0)
    out_pre = _sc_conv(flat_pad, grid_ext, h)
    s, q = _bn_reduce(out_pre)
    gamma8 = jnp.broadcast_to(gamma.reshape(1, C), (8, C))
    beta8 = jnp.broadcast_to(beta.reshape(1, C), (8, C))
    y = _bn_norm(out_pre, s, q, gamma8, beta8)
    return y.reshape(B, N, C)
